# Initial kernel scaffold; baseline (speedup 1.0000x reference)
import jax
import jax.numpy as jnp
from jax.experimental import pallas as pl


def kernel(x_solute, x_solvent, edge_index_solute, edge_index_solvent, graph_ids, W_in_solute, W_msg_solute, W_in_solvent, W_msg_solvent, W1, b1, W2, b2, W3, b3):
    raise NotImplementedError("write your pallas kernel here")


# speedup vs baseline: 6.7978x; 6.7978x over previous
import functools

import jax
import jax.numpy as jnp
from jax import lax
from jax.experimental import pallas as pl
from jax.experimental.pallas import tpu as pltpu
from jax.experimental.pallas import tpu_sc as plsc

N = 10000
NP_ = 10240
E = 320000
D = 128
H = 128
HH = H // 2
B = 128
DEPTH = 3

NC = 2
NS = 16
K = 80
NCH = E // (NS * K)
R = 5
NSUP = NCH // R
RPT = NP_ // NS


def _sc_body(h4, src3, dst3, zrows, m4, sidx, didx, rows, msh, sems):
    c = lax.axis_index("c")
    s = lax.axis_index("s")
    row0 = s * RPT
    for mpn in range(2):
        pltpu.sync_copy(zrows, msh.at[pl.ds(row0, RPT)])
        pltpu.sync_copy(src3.at[mpn, s], sidx)
        pltpu.sync_copy(dst3.at[mpn, s], didx)
        plsc.subcore_barrier()
        hv = h4.at[mpn, c]
        for r in range(R):
            pltpu.async_copy(hv.at[sidx.at[r]], rows.at[r], sems.at[r])

        def loop(jj, carry, hv=hv):
            for r in range(R):
                j = jj * R + r
                pltpu.make_async_copy(hv.at[sidx.at[j]], rows.at[r],
                                      sems.at[r]).wait()
                pltpu.sync_copy(rows.at[r], msh.at[didx.at[j]], add=True)

                @pl.when(jj < NSUP - 1)
                def _():
                    pltpu.async_copy(hv.at[sidx.at[j + R]], rows.at[r],
                                     sems.at[r])
            return carry

        lax.fori_loop(0, NSUP, loop, 0)
        plsc.subcore_barrier()
        pltpu.sync_copy(msh.at[pl.ds(row0, RPT)],
                        m4.at[mpn, c].at[pl.ds(row0, RPT)])


_sc_segsum = functools.partial(
    pl.kernel,
    out_type=jax.ShapeDtypeStruct((2, NC, NP_, HH), jnp.float32),
    mesh=plsc.VectorSubcoreMesh(core_axis_name="c", subcore_axis_name="s",
                                num_cores=NC, num_subcores=NS),
    scratch_types=[
        pltpu.VMEM((NCH, K), jnp.int32),
        pltpu.VMEM((NCH, K), jnp.int32),
        pltpu.VMEM((R, K, HH), jnp.float32),
        pltpu.VMEM_SHARED((NP_, HH), jnp.float32),
        pltpu.SemaphoreType.DMA((R,)),
    ],
    compiler_params=pltpu.CompilerParams(use_tc_tiling_on_sc=False),
)(_sc_body)


_BN = 1024


def _mm_relu_body(x_ref, w_ref, o_ref):
    res = jnp.maximum(
        jnp.dot(x_ref[0], w_ref[0], preferred_element_type=jnp.float32,
                precision=lax.Precision.HIGHEST), 0.0)
    o_ref[0, 0] = res[:, :HH]
    o_ref[0, 1] = res[:, HH:]


def _mm_add_relu_body(m_ref, w_ref, a_ref, o_ref):
    m = jnp.concatenate([m_ref[0, 0], m_ref[0, 1]], axis=1)
    a = jnp.concatenate([a_ref[0, 0], a_ref[0, 1]], axis=1)
    res = jnp.maximum(
        jnp.dot(m, w_ref[0], preferred_element_type=jnp.float32,
                precision=lax.Precision.HIGHEST) + a, 0.0)
    o_ref[0, 0] = res[:, :HH]
    o_ref[0, 1] = res[:, HH:]


def _tc_in(x2, w2):
    return pl.pallas_call(
        _mm_relu_body,
        grid=(2, NP_ // _BN),
        in_specs=[
            pl.BlockSpec((1, _BN, D), lambda c, i: (c, i, 0)),
            pl.BlockSpec((1, D, H), lambda c, i: (c, 0, 0)),
        ],
        out_specs=pl.BlockSpec((1, NC, _BN, HH), lambda c, i: (c, 0, i, 0)),
        out_shape=jax.ShapeDtypeStruct((2, NC, NP_, HH), jnp.float32),
    )(x2, w2)


def _tc_update(m4, w2, h04):
    return pl.pallas_call(
        _mm_add_relu_body,
        grid=(2, NP_ // _BN),
        in_specs=[
            pl.BlockSpec((1, NC, _BN, HH), lambda c, i: (c, 0, i, 0)),
            pl.BlockSpec((1, H, H), lambda c, i: (c, 0, 0)),
            pl.BlockSpec((1, NC, _BN, HH), lambda c, i: (c, 0, i, 0)),
        ],
        out_specs=pl.BlockSpec((1, NC, _BN, HH), lambda c, i: (c, 0, i, 0)),
        out_shape=jax.ShapeDtypeStruct((2, NC, NP_, HH), jnp.float32),
    )(m4, w2, h04)


def _seg_body(ids_ref, h_ref, o_ref):
    i = pl.program_id(1)
    oh = (lax.broadcasted_iota(jnp.int32, (B, _BN), 0)
          == ids_ref[0]).astype(jnp.float32)
    h = jnp.concatenate([h_ref[0, 0], h_ref[0, 1]], axis=1)
    part = jnp.dot(oh, h, preferred_element_type=jnp.float32,
                precision=lax.Precision.HIGHEST)

    @pl.when(i == 0)
    def _():
        o_ref[0] = part

    @pl.when(i > 0)
    def _():
        o_ref[0] = o_ref[0] + part


def _tc_segsums(ids3, h4):
    return pl.pallas_call(
        _seg_body,
        grid=(2, NP_ // _BN),
        in_specs=[
            pl.BlockSpec((1, 1, _BN), lambda c, i: (i, 0, 0)),
            pl.BlockSpec((1, NC, _BN, HH), lambda c, i: (c, 0, i, 0)),
        ],
        out_specs=pl.BlockSpec((1, B, H), lambda c, i: (c, 0, 0)),
        out_shape=jax.ShapeDtypeStruct((2, B, H), jnp.float32),
    )(ids3, h4)


def _mlp_body(sums_ref, ids_ref, w1_ref, b1_ref, w2_ref, b2_ref, w3_ref,
              b3_ref, o_ref):
    cnt = jnp.zeros((B, 1), jnp.float32)
    for i in range(NP_ // _BN):
        m = (lax.broadcasted_iota(jnp.int32, (B, _BN), 0)
             == ids_ref[pl.ds(i, 1), :]).astype(jnp.float32)
        cnt = cnt + jnp.sum(m, axis=1, keepdims=True)
    cnt = jnp.maximum(cnt, 1.0)
    vec = jnp.concatenate([sums_ref[0] / cnt, sums_ref[1] / cnt], axis=1)
    hh = jnp.maximum(
        jnp.dot(vec, w1_ref[...], preferred_element_type=jnp.float32,
                precision=lax.Precision.HIGHEST)
        + b1_ref[...], 0.0)
    hh = jnp.maximum(
        jnp.dot(hh, w2_ref[...], preferred_element_type=jnp.float32,
                precision=lax.Precision.HIGHEST)
        + b2_ref[...], 0.0)
    o_ref[...] = (jnp.dot(hh, w3_ref[...], preferred_element_type=jnp.float32,
                precision=lax.Precision.HIGHEST)
                  + b3_ref[...])


def _tc_mlp(sums, ids_r, w1, b1, w2, b2, w3, b3):
    return pl.pallas_call(
        _mlp_body,
        out_shape=jax.ShapeDtypeStruct((B, 1), jnp.float32),
    )(sums, ids_r, w1, b1, w2, b2, w3, b3)


def kernel(x_solute, x_solvent, edge_index_solute, edge_index_solvent,
           graph_ids, W_in_solute, W_msg_solute, W_in_solvent, W_msg_solvent,
           W1, b1, W2, b2, W3, b3):
    x2 = jnp.zeros((2, NP_, D), jnp.float32).at[:, :N].set(
        jnp.stack([x_solute, x_solvent]))
    win2 = jnp.stack([W_in_solute, W_in_solvent])
    wm2 = jnp.stack([W_msg_solute, W_msg_solvent])
    src3 = jnp.stack([edge_index_solute[0].reshape(NS, NCH, K),
                      edge_index_solvent[0].reshape(NS, NCH, K)])
    dst3 = jnp.stack([edge_index_solute[1].reshape(NS, NCH, K),
                      edge_index_solvent[1].reshape(NS, NCH, K)])
    zrows = jnp.zeros((RPT, HH), jnp.float32)
    ids_pad = jnp.full((NP_,), B, jnp.int32).at[:N].set(graph_ids)
    ids_r = ids_pad.reshape(NP_ // _BN, _BN)
    ids3 = ids_pad.reshape(NP_ // _BN, 1, _BN)

    h0 = _tc_in(x2, win2)
    h = h0
    for _ in range(DEPTH):
        m = _sc_segsum(h, src3, dst3, zrows)
        h = _tc_update(m, wm2, h0)
    sums = _tc_segsums(ids3, h)
    return _tc_mlp(sums, ids_r, W1, b1.reshape(1, -1), W2, b2.reshape(1, -1),
                   W3, b3.reshape(1, 1))


# speedup vs baseline: 6.8928x; 1.0140x over previous
import functools

import jax
import jax.numpy as jnp
from jax import lax
from jax.experimental import pallas as pl
from jax.experimental.pallas import tpu as pltpu
from jax.experimental.pallas import tpu_sc as plsc

N = 10000
NP_ = 10240
E = 320000
D = 128
H = 128
HH = H // 2
B = 128
DEPTH = 3

NC = 2
NS = 16
K = 80
NCH = E // (NS * K)
R = 5
NSUP = NCH // R
RPT = NP_ // NS


def _sc_body(h4, src3, dst3, zrows, m4, sidx, didx, rows, msh, sems, ssems):
    c = lax.axis_index("c")
    s = lax.axis_index("s")
    row0 = s * RPT
    for mpn in range(2):
        pltpu.sync_copy(zrows, msh.at[pl.ds(row0, RPT)])
        pltpu.sync_copy(src3.at[mpn, s], sidx)
        pltpu.sync_copy(dst3.at[mpn, s], didx)
        plsc.subcore_barrier()
        hv = h4.at[mpn, c]
        for r in range(R):
            pltpu.async_copy(hv.at[sidx.at[r]], rows.at[r], sems.at[r])

        def loop(jj, carry, hv=hv):
            for r in range(R):
                j = jj * R + r
                prev = (r - 1) % R
                pltpu.make_async_copy(hv.at[sidx.at[j]], rows.at[r],
                                      sems.at[r]).wait()

                @pl.when(j > 0)
                def _(j=j, prev=prev):
                    pltpu.make_async_copy(rows.at[prev],
                                          msh.at[didx.at[j - 1]],
                                          ssems.at[prev]).wait()

                    @pl.when(j - 1 + R < NCH)
                    def _():
                        pltpu.async_copy(hv.at[sidx.at[j - 1 + R]],
                                         rows.at[prev], sems.at[prev])

                pltpu.async_copy(rows.at[r], msh.at[didx.at[j]], ssems.at[r],
                                 add=True)
            return carry

        lax.fori_loop(0, NSUP, loop, 0)
        pltpu.make_async_copy(rows.at[R - 1], msh.at[didx.at[NCH - 1]],
                              ssems.at[R - 1]).wait()
        plsc.subcore_barrier()
        pltpu.sync_copy(msh.at[pl.ds(row0, RPT)],
                        m4.at[mpn, c].at[pl.ds(row0, RPT)])


_sc_segsum = functools.partial(
    pl.kernel,
    out_type=jax.ShapeDtypeStruct((2, NC, NP_, HH), jnp.float32),
    mesh=plsc.VectorSubcoreMesh(core_axis_name="c", subcore_axis_name="s",
                                num_cores=NC, num_subcores=NS),
    scratch_types=[
        pltpu.VMEM((NCH, K), jnp.int32),
        pltpu.VMEM((NCH, K), jnp.int32),
        pltpu.VMEM((R, K, HH), jnp.float32),
        pltpu.VMEM_SHARED((NP_, HH), jnp.float32),
        pltpu.SemaphoreType.DMA((R,)),
        pltpu.SemaphoreType.DMA((R,)),
    ],
    compiler_params=pltpu.CompilerParams(use_tc_tiling_on_sc=False),
)(_sc_body)


_BN = 1024


def _mm_relu_body(x_ref, w_ref, o_ref):
    res = jnp.maximum(
        jnp.dot(x_ref[0], w_ref[0], preferred_element_type=jnp.float32), 0.0)
    o_ref[0, 0] = res[:, :HH]
    o_ref[0, 1] = res[:, HH:]


def _mm_add_relu_body(m_ref, w_ref, a_ref, o_ref):
    m = jnp.concatenate([m_ref[0, 0], m_ref[0, 1]], axis=1)
    a = jnp.concatenate([a_ref[0, 0], a_ref[0, 1]], axis=1)
    res = jnp.maximum(
        jnp.dot(m, w_ref[0], preferred_element_type=jnp.float32) + a, 0.0)
    o_ref[0, 0] = res[:, :HH]
    o_ref[0, 1] = res[:, HH:]


def _tc_in(x2, w2):
    return pl.pallas_call(
        _mm_relu_body,
        grid=(2, NP_ // _BN),
        in_specs=[
            pl.BlockSpec((1, _BN, D), lambda c, i: (c, i, 0)),
            pl.BlockSpec((1, D, H), lambda c, i: (c, 0, 0)),
        ],
        out_specs=pl.BlockSpec((1, NC, _BN, HH), lambda c, i: (c, 0, i, 0)),
        out_shape=jax.ShapeDtypeStruct((2, NC, NP_, HH), jnp.float32),
    )(x2, w2)


def _tc_update(m4, w2, h04):
    return pl.pallas_call(
        _mm_add_relu_body,
        grid=(2, NP_ // _BN),
        in_specs=[
            pl.BlockSpec((1, NC, _BN, HH), lambda c, i: (c, 0, i, 0)),
            pl.BlockSpec((1, H, H), lambda c, i: (c, 0, 0)),
            pl.BlockSpec((1, NC, _BN, HH), lambda c, i: (c, 0, i, 0)),
        ],
        out_specs=pl.BlockSpec((1, NC, _BN, HH), lambda c, i: (c, 0, i, 0)),
        out_shape=jax.ShapeDtypeStruct((2, NC, NP_, HH), jnp.float32),
    )(m4, w2, h04)


def _seg_body(ids_ref, h_ref, o_ref):
    i = pl.program_id(1)
    oh = (lax.broadcasted_iota(jnp.int32, (B, _BN), 0)
          == ids_ref[0]).astype(jnp.float32)
    h = jnp.concatenate([h_ref[0, 0], h_ref[0, 1]], axis=1)
    part = jnp.dot(oh, h, preferred_element_type=jnp.float32,
                precision=lax.Precision.HIGHEST)

    @pl.when(i == 0)
    def _():
        o_ref[0] = part

    @pl.when(i > 0)
    def _():
        o_ref[0] = o_ref[0] + part


def _tc_segsums(ids3, h4):
    return pl.pallas_call(
        _seg_body,
        grid=(2, NP_ // _BN),
        in_specs=[
            pl.BlockSpec((1, 1, _BN), lambda c, i: (i, 0, 0)),
            pl.BlockSpec((1, NC, _BN, HH), lambda c, i: (c, 0, i, 0)),
        ],
        out_specs=pl.BlockSpec((1, B, H), lambda c, i: (c, 0, 0)),
        out_shape=jax.ShapeDtypeStruct((2, B, H), jnp.float32),
    )(ids3, h4)


def _mlp_body(sums_ref, ids_ref, w1_ref, b1_ref, w2_ref, b2_ref, w3_ref,
              b3_ref, o_ref):
    cnt = jnp.zeros((B, 1), jnp.float32)
    for i in range(NP_ // _BN):
        m = (lax.broadcasted_iota(jnp.int32, (B, _BN), 0)
             == ids_ref[pl.ds(i, 1), :]).astype(jnp.float32)
        cnt = cnt + jnp.sum(m, axis=1, keepdims=True)
    cnt = jnp.maximum(cnt, 1.0)
    vec = jnp.concatenate([sums_ref[0] / cnt, sums_ref[1] / cnt], axis=1)
    hh = jnp.maximum(
        jnp.dot(vec, w1_ref[...], preferred_element_type=jnp.float32)
        + b1_ref[...], 0.0)
    hh = jnp.maximum(
        jnp.dot(hh, w2_ref[...], preferred_element_type=jnp.float32)
        + b2_ref[...], 0.0)
    o_ref[...] = (jnp.dot(hh, w3_ref[...], preferred_element_type=jnp.float32)
                  + b3_ref[...])


def _tc_mlp(sums, ids_r, w1, b1, w2, b2, w3, b3):
    return pl.pallas_call(
        _mlp_body,
        out_shape=jax.ShapeDtypeStruct((B, 1), jnp.float32),
    )(sums, ids_r, w1, b1, w2, b2, w3, b3)


def kernel(x_solute, x_solvent, edge_index_solute, edge_index_solvent,
           graph_ids, W_in_solute, W_msg_solute, W_in_solvent, W_msg_solvent,
           W1, b1, W2, b2, W3, b3):
    x2 = jnp.zeros((2, NP_, D), jnp.float32).at[:, :N].set(
        jnp.stack([x_solute, x_solvent]))
    win2 = jnp.stack([W_in_solute, W_in_solvent])
    wm2 = jnp.stack([W_msg_solute, W_msg_solvent])
    src3 = jnp.stack([edge_index_solute[0].reshape(NS, NCH, K),
                      edge_index_solvent[0].reshape(NS, NCH, K)])
    dst3 = jnp.stack([edge_index_solute[1].reshape(NS, NCH, K),
                      edge_index_solvent[1].reshape(NS, NCH, K)])
    zrows = jnp.zeros((RPT, HH), jnp.float32)
    ids_pad = jnp.full((NP_,), B, jnp.int32).at[:N].set(graph_ids)
    ids_r = ids_pad.reshape(NP_ // _BN, _BN)
    ids3 = ids_pad.reshape(NP_ // _BN, 1, _BN)

    h0 = _tc_in(x2, win2)
    h = h0
    for _ in range(DEPTH):
        m = _sc_segsum(h, src3, dst3, zrows)
        h = _tc_update(m, wm2, h0)
    sums = _tc_segsums(ids3, h)
    return _tc_mlp(sums, ids_r, W1, b1.reshape(1, -1), W2, b2.reshape(1, -1),
                   W3, b3.reshape(1, 1))


# speedup vs baseline: 8.0622x; 1.1697x over previous
import functools

import jax
import jax.numpy as jnp
from jax import lax
from jax.experimental import pallas as pl
from jax.experimental.pallas import tpu as pltpu
from jax.experimental.pallas import tpu_sc as plsc

N = 10000
NP_ = 10240
E = 320000
D = 128
H = 128
HH = H // 2
B = 128
DEPTH = 3

NC = 2
NS = 16
K = 80
NCH = E // (NS * K)
R = 5
NSUP = NCH // R
RPT = NP_ // NS


def _sc_body(h2, src3, dst3, zrows, m2, sidx, didx, rows, msh, sems, ssems):
    c = lax.axis_index("c")
    s = lax.axis_index("s")
    row0 = s * RPT
    pltpu.sync_copy(zrows, msh.at[pl.ds(row0, RPT)])
    pltpu.sync_copy(src3.at[s], sidx)
    pltpu.sync_copy(dst3.at[s], didx)
    plsc.subcore_barrier()
    hv = h2.at[c]
    for r in range(R):
        pltpu.async_copy(hv.at[sidx.at[r]], rows.at[r], sems.at[r])

    def loop(jj, carry):
        for r in range(R):
            j = jj * R + r
            prev = (r - 1) % R
            pltpu.make_async_copy(hv.at[sidx.at[j]], rows.at[r],
                                  sems.at[r]).wait()

            @pl.when(j > 0)
            def _(j=j, prev=prev):
                pltpu.make_async_copy(rows.at[prev],
                                      msh.at[didx.at[j - 1]],
                                      ssems.at[prev]).wait()

                @pl.when(j - 1 + R < NCH)
                def _():
                    pltpu.async_copy(hv.at[sidx.at[j - 1 + R]],
                                     rows.at[prev], sems.at[prev])

            pltpu.async_copy(rows.at[r], msh.at[didx.at[j]], ssems.at[r],
                             add=True)
        return carry

    lax.fori_loop(0, NSUP, loop, 0)
    pltpu.make_async_copy(rows.at[R - 1], msh.at[didx.at[NCH - 1]],
                          ssems.at[R - 1]).wait()
    plsc.subcore_barrier()
    pltpu.sync_copy(msh.at[pl.ds(row0, RPT)],
                    m2.at[c].at[pl.ds(row0, RPT)])


_sc_segsum = functools.partial(
    pl.kernel,
    out_type=jax.ShapeDtypeStruct((NC, NP_, HH), jnp.float32),
    mesh=plsc.VectorSubcoreMesh(core_axis_name="c", subcore_axis_name="s",
                                num_cores=NC, num_subcores=NS),
    scratch_types=[
        pltpu.VMEM((NCH, K), jnp.int32),
        pltpu.VMEM((NCH, K), jnp.int32),
        pltpu.VMEM((R, K, HH), jnp.float32),
        pltpu.VMEM_SHARED((NP_, HH), jnp.float32),
        pltpu.SemaphoreType.DMA((R,)),
        pltpu.SemaphoreType.DMA((R,)),
    ],
    compiler_params=pltpu.CompilerParams(use_tc_tiling_on_sc=False),
)(_sc_body)


_BN = 1024


def _mm_relu_body(x_ref, w_ref, o_ref):
    res = jnp.maximum(
        jnp.dot(x_ref[...], w_ref[...], preferred_element_type=jnp.float32),
        0.0)
    o_ref[0] = res[:, :HH]
    o_ref[1] = res[:, HH:]


def _mm_add_relu_body(m_ref, w_ref, a_ref, o_ref):
    m = jnp.concatenate([m_ref[0], m_ref[1]], axis=1)
    a = jnp.concatenate([a_ref[0], a_ref[1]], axis=1)
    res = jnp.maximum(
        jnp.dot(m, w_ref[...], preferred_element_type=jnp.float32) + a, 0.0)
    o_ref[0] = res[:, :HH]
    o_ref[1] = res[:, HH:]


def _tc_in(xp, w):
    return pl.pallas_call(
        _mm_relu_body,
        grid=(NP_ // _BN,),
        in_specs=[
            pl.BlockSpec((_BN, D), lambda i: (i, 0)),
            pl.BlockSpec((D, H), lambda i: (0, 0)),
        ],
        out_specs=pl.BlockSpec((NC, _BN, HH), lambda i: (0, i, 0)),
        out_shape=jax.ShapeDtypeStruct((NC, NP_, HH), jnp.float32),
    )(xp, w)


def _tc_update(m2, w, h02):
    return pl.pallas_call(
        _mm_add_relu_body,
        grid=(NP_ // _BN,),
        in_specs=[
            pl.BlockSpec((NC, _BN, HH), lambda i: (0, i, 0)),
            pl.BlockSpec((H, H), lambda i: (0, 0)),
            pl.BlockSpec((NC, _BN, HH), lambda i: (0, i, 0)),
        ],
        out_specs=pl.BlockSpec((NC, _BN, HH), lambda i: (0, i, 0)),
        out_shape=jax.ShapeDtypeStruct((NC, NP_, HH), jnp.float32),
    )(m2, w, h02)


def _seg_body(ids_ref, h_ref, o_ref):
    i = pl.program_id(0)
    oh = (lax.broadcasted_iota(jnp.int32, (B, _BN), 0)
          == ids_ref[0]).astype(jnp.float32)
    h = jnp.concatenate([h_ref[0], h_ref[1]], axis=1)
    part = jnp.dot(oh, h, preferred_element_type=jnp.float32,
                   precision=lax.Precision.HIGHEST)

    @pl.when(i == 0)
    def _():
        o_ref[...] = part

    @pl.when(i > 0)
    def _():
        o_ref[...] = o_ref[...] + part


def _tc_segsums(ids3, h2):
    return pl.pallas_call(
        _seg_body,
        grid=(NP_ // _BN,),
        in_specs=[
            pl.BlockSpec((1, 1, _BN), lambda i: (i, 0, 0)),
            pl.BlockSpec((NC, _BN, HH), lambda i: (0, i, 0)),
        ],
        out_specs=pl.BlockSpec((B, H), lambda i: (0, 0)),
        out_shape=jax.ShapeDtypeStruct((B, H), jnp.float32),
    )(ids3, h2)


def _mlp_body(s0_ref, s1_ref, ids_ref, w1_ref, b1_ref, w2_ref, b2_ref,
              w3_ref, b3_ref, o_ref):
    cnt = jnp.zeros((B, 1), jnp.float32)
    for i in range(NP_ // _BN):
        m = (lax.broadcasted_iota(jnp.int32, (B, _BN), 0)
             == ids_ref[pl.ds(i, 1), :]).astype(jnp.float32)
        cnt = cnt + jnp.sum(m, axis=1, keepdims=True)
    cnt = jnp.maximum(cnt, 1.0)
    vec = jnp.concatenate([s0_ref[...] / cnt, s1_ref[...] / cnt], axis=1)
    hh = jnp.maximum(
        jnp.dot(vec, w1_ref[...], preferred_element_type=jnp.float32)
        + b1_ref[...], 0.0)
    hh = jnp.maximum(
        jnp.dot(hh, w2_ref[...], preferred_element_type=jnp.float32)
        + b2_ref[...], 0.0)
    o_ref[...] = (jnp.dot(hh, w3_ref[...], preferred_element_type=jnp.float32)
                  + b3_ref[...])


def _tc_mlp(s0, s1, ids_r, w1, b1, w2, b2, w3, b3):
    return pl.pallas_call(
        _mlp_body,
        out_shape=jax.ShapeDtypeStruct((B, 1), jnp.float32),
    )(s0, s1, ids_r, w1, b1, w2, b2, w3, b3)


def kernel(x_solute, x_solvent, edge_index_solute, edge_index_solvent,
           graph_ids, W_in_solute, W_msg_solute, W_in_solvent, W_msg_solvent,
           W1, b1, W2, b2, W3, b3):
    zrows = jnp.zeros((RPT, HH), jnp.float32)
    ids_pad = jnp.full((NP_,), B, jnp.int32).at[:N].set(graph_ids)
    ids_r = ids_pad.reshape(NP_ // _BN, _BN)
    ids3 = ids_pad.reshape(NP_ // _BN, 1, _BN)

    sums = []
    for x, ei, w_in, w_msg in (
            (x_solute, edge_index_solute, W_in_solute, W_msg_solute),
            (x_solvent, edge_index_solvent, W_in_solvent, W_msg_solvent)):
        xp = jnp.zeros((NP_, D), jnp.float32).at[:N].set(x)
        src3 = ei[0].reshape(NS, NCH, K)
        dst3 = ei[1].reshape(NS, NCH, K)
        h0 = _tc_in(xp, w_in)
        h = h0
        for _ in range(DEPTH):
            m = _sc_segsum(h, src3, dst3, zrows)
            h = _tc_update(m, w_msg, h0)
        sums.append(_tc_segsums(ids3, h))
    return _tc_mlp(sums[0], sums[1], ids_r, W1, b1.reshape(1, -1),
                   W2, b2.reshape(1, -1), W3, b3.reshape(1, 1))
